# baseline (speedup 1.0000x reference)
import functools

import jax
import jax.numpy as jnp
from jax import lax
from jax.experimental import pallas as pl
from jax.experimental.pallas import tpu as pltpu
from jax.experimental.pallas import tpu_sc as plsc

N = 10000
E = 320000
D = 128
HD = D // 2
N_PAD = 10240
NC, NS = 2, 16
NW = NC * NS
B = 96
E_PAD = 322560
EPT = E_PAD // NS
NB = EPT // B
NB2 = E_PAD // NW // B
NSLOT = 4
RPT = N_PAD // NS
R = 2048
GRID = N_PAD // R

f32 = jnp.float32


@functools.lru_cache(maxsize=None)
def _sc_kernels():
  mesh = plsc.VectorSubcoreMesh(core_axis_name="c", subcore_axis_name="s")

  @functools.partial(
      pl.kernel,
      out_type=jax.ShapeDtypeStruct((NW, N_PAD), f32),
      mesh=mesh,
      scratch_types=[
          pltpu.VMEM((NB2, B), jnp.int32),
          pltpu.VMEM((N_PAD,), f32),
      ],
      compiler_params=pltpu.CompilerParams(needs_layout_passes=False),
  )
  def deg_kernel(dst_hbm, out_hbm, dbuf, hist):
    cid = lax.axis_index("c")
    sid = lax.axis_index("s")
    gid = cid * NS + sid
    pltpu.sync_copy(dst_hbm.at[gid], dbuf)
    zv = jnp.zeros((16,), f32)

    def zbody(i, c):
      hist[pl.ds(pl.multiple_of(i * 16, 16), 16)] = zv
      return c

    lax.fori_loop(jnp.int32(0), jnp.int32(N_PAD // 16), zbody, jnp.int32(0))
    ones = jnp.ones((16,), f32)

    def ebody(j, c):
      def kbody(k, c2):
        idx = dbuf[j, pl.ds(pl.multiple_of(k * 16, 16), 16)]
        plsc.addupdate_scatter(hist, [idx], ones)
        return c2

      return lax.fori_loop(jnp.int32(0), jnp.int32(B // 16), kbody, c)

    lax.fori_loop(jnp.int32(0), jnp.int32(NB2), ebody, jnp.int32(0))
    pltpu.sync_copy(hist, out_hbm.at[gid])

  @functools.partial(
      pl.kernel,
      out_type=jax.ShapeDtypeStruct((NC, N_PAD, HD), f32),
      mesh=mesh,
      scratch_types=[
          pltpu.VMEM((NB, B), jnp.int32),
          pltpu.VMEM((NB, B), jnp.int32),
          [pltpu.VMEM((B, HD), f32)] * 2,
          pltpu.VMEM_SHARED((N_PAD, HD), f32),
          [pltpu.SemaphoreType.DMA] * 2,
      ],
      compiler_params=pltpu.CompilerParams(use_tc_tiling_on_sc=False),
  )
  def edge_kernel(hws_hbm, src_hbm, dst_hbm, out,
                  sbuf, dbuf, rows, acc_sh, gsem):
    cid = lax.axis_index("c")
    sid = lax.axis_index("s")
    row0 = sid * RPT

    zv = jnp.zeros((16,), f32)

    def zbody(i, c):
      r = i // (HD // 16)
      col = (i % (HD // 16)) * 16
      rows[0][r, pl.ds(pl.multiple_of(col, 16), 16)] = zv
      return c

    lax.fori_loop(jnp.int32(0), jnp.int32(B * HD // 16), zbody, jnp.int32(0))
    for k in range(RPT // B):
      pltpu.sync_copy(rows[0], acc_sh.at[pl.ds(row0 + k * B, B)])

    pltpu.sync_copy(src_hbm.at[sid], sbuf)
    pltpu.sync_copy(dst_hbm.at[sid], dbuf)
    plsc.subcore_barrier()

    table = hws_hbm.at[cid]

    def gather(j, k):
      j = jnp.int32(j)
      return pltpu.make_async_copy(table.at[sbuf.at[j]], rows[k], gsem[k])

    def scatter(j, k):
      j = jnp.int32(j)
      pltpu.sync_copy(rows[k], acc_sh.at[dbuf.at[j]], add=True)

    gather(0, 0).start()

    def pbody(i, c):
      ja = i * 2
      jb = ja + 1
      gather(jb, 1).start()
      gather(ja, 0).wait()
      scatter(ja, 0)
      gather(ja + 2, 0).start()
      gather(jb, 1).wait()
      scatter(jb, 1)
      return c

    lax.fori_loop(jnp.int32(0), jnp.int32((NB - 2) // 2), pbody, jnp.int32(0))
    gather(NB - 1, 1).start()
    gather(NB - 2, 0).wait()
    scatter(NB - 2, 0)
    gather(NB - 1, 1).wait()
    scatter(NB - 1, 1)

    plsc.subcore_barrier()
    pltpu.sync_copy(acc_sh.at[pl.ds(row0, RPT)], out.at[cid, pl.ds(row0, RPT)])

  return deg_kernel, edge_kernel


def _ln(t, g, b):
  m = jnp.mean(t, axis=-1, keepdims=True)
  c = t - m
  v = jnp.mean(c * c, axis=-1, keepdims=True)
  return c * lax.rsqrt(v + 1e-5) * g + b


def _elu(t):
  return jnp.where(t > 0, t, jnp.exp(jnp.minimum(t, 0.0)) - 1.0)


def _dot(a, b):
  return jnp.dot(a, b, preferred_element_type=f32,
                 precision=lax.Precision.HIGHEST)


def _split_store(out_ref, t):
  out_ref[0] = t[:, :HD]
  out_ref[1] = t[:, HD:]


def _unsplit(ref):
  return jnp.concatenate([ref[0], ref[1]], axis=-1)


def _enc_body(x_ref, win_ref, bin_ref, wg0_ref, hist_ref, hws_ref, dis_ref):
  h = jnp.maximum(_dot(x_ref[...], win_ref[...]) + bin_ref[...], 0.0)
  deg = jnp.sum(hist_ref[...], axis=1, keepdims=True) + 1.0
  dis = lax.rsqrt(deg)
  dis_ref[...] = dis
  _split_store(hws_ref, _dot(h, wg0_ref[...]) * dis)


def _mid_body(acc_ref, hws_ref, dis_ref, bg_ref, lng_ref, lnb_ref,
              wg1_ref, out_ref):
  dis = dis_ref[...]
  t = (_unsplit(acc_ref) + _unsplit(hws_ref)) * dis + bg_ref[...]
  h = _elu(_ln(t, lng_ref[...], lnb_ref[...]))
  _split_store(out_ref, _dot(h, wg1_ref[...]) * dis)


def _tail_body(acc_ref, hws_ref, dis_ref, bg_ref, lng_ref, lnb_ref,
               wout_ref, bout_ref, ln2g_ref, ln2b_ref, w1_ref, b1_ref,
               w2t_ref, b2_ref, w3t_ref, b3_ref, out_ref):
  t = (_unsplit(acc_ref) + _unsplit(hws_ref)) * dis_ref[...] + bg_ref[...]
  h = _elu(_ln(t, lng_ref[...], lnb_ref[...]))
  ne = _ln(jnp.maximum(_dot(h, wout_ref[...]) + bout_ref[...], 0.0),
           ln2g_ref[...], ln2b_ref[...])
  y = jnp.maximum(_dot(ne, w1_ref[...]) + b1_ref[...], 0.0)
  w2t = w2t_ref[...]
  cols = [jnp.sum(y * w2t[k:k + 1, :], axis=1, keepdims=True)
          for k in range(w2t.shape[0])]
  y = jnp.maximum(jnp.concatenate(cols, axis=1) + b2_ref[...], 0.0)
  out_ref[...] = jnp.sum(y * w3t_ref[...], axis=1, keepdims=True) + b3_ref[...]


def _row_spec(w):
  return pl.BlockSpec((R, w), lambda i: (i, jnp.int32(0)))


def _split_spec():
  return pl.BlockSpec((2, R, HD), lambda i: (jnp.int32(0), i, jnp.int32(0)))


def _full_spec(h, w):
  return pl.BlockSpec((h, w), lambda i: (jnp.int32(0), jnp.int32(0)))


_SPLIT_SHAPE = jax.ShapeDtypeStruct((2, N_PAD, HD), f32)


def _enc_call(x_pad, w_in, b_in, wg0, hist_t):
  return pl.pallas_call(
      _enc_body,
      grid=(GRID,),
      in_specs=[
          _row_spec(D), _full_spec(D, D), _full_spec(1, D), _full_spec(D, D),
          _row_spec(NW),
      ],
      out_specs=[_split_spec(), _row_spec(1)],
      out_shape=[_SPLIT_SHAPE, jax.ShapeDtypeStruct((N_PAD, 1), f32)],
  )(x_pad, w_in, b_in, wg0, hist_t)


def _mid_call(acc, hws, dis, bg, lng, lnb, wg1):
  return pl.pallas_call(
      _mid_body,
      grid=(GRID,),
      in_specs=[
          _split_spec(), _split_spec(), _row_spec(1),
          _full_spec(1, D), _full_spec(1, D), _full_spec(1, D),
          _full_spec(D, D),
      ],
      out_specs=_split_spec(),
      out_shape=_SPLIT_SHAPE,
  )(acc, hws, dis, bg, lng, lnb, wg1)


def _tail_call(acc, hws, dis, bg, lng, lnb, wout, bout, ln2g, ln2b,
               w1, b1, w2t, b2, w3t, b3):
  h1 = w1.shape[1]
  h2 = w2t.shape[0]
  return pl.pallas_call(
      _tail_body,
      grid=(GRID,),
      in_specs=[
          _split_spec(), _split_spec(), _row_spec(1),
          _full_spec(1, D), _full_spec(1, D), _full_spec(1, D),
          _full_spec(D, D), _full_spec(1, D),
          _full_spec(1, D), _full_spec(1, D),
          _full_spec(D, h1), _full_spec(1, h1),
          _full_spec(h2, h1), _full_spec(1, h2),
          _full_spec(1, h2), _full_spec(1, 1),
      ],
      out_specs=_row_spec(1),
      out_shape=jax.ShapeDtypeStruct((N_PAD, 1), f32),
  )(acc, hws, dis, bg, lng, lnb, wout, bout, ln2g, ln2b,
    w1, b1, w2t, b2, w3t, b3)


def kernel(x, edge_index, W_in, b_in, Wg0, bg0, ln0_g, ln0_b, Wg1, bg1,
           ln1_g, ln1_b, W_out, b_out, ln2_g, ln2_b, W1, b1, W2, b2, W3, b3):
  deg_kernel, edge_kernel = _sc_kernels()

  ei = edge_index.astype(jnp.int32)
  npad = E_PAD - E
  src_p = jnp.concatenate([ei[0], jnp.zeros((npad,), jnp.int32)])
  dst_p = jnp.concatenate([ei[1], jnp.full((npad,), N_PAD - 1, jnp.int32)])
  src3 = src_p.reshape(NS, NB, B)
  dst3 = dst_p.reshape(NS, NB, B)
  dst3d = dst_p.reshape(NW, NB2, B)
  x_pad = jnp.pad(x.astype(f32), ((0, N_PAD - N), (0, 0)))

  row = lambda a: a.astype(f32).reshape(1, -1)
  b_in_, bg0_, bg1_, b_out_ = row(b_in), row(bg0), row(bg1), row(b_out)
  ln0g_, ln0b_ = row(ln0_g), row(ln0_b)
  ln1g_, ln1b_ = row(ln1_g), row(ln1_b)
  ln2g_, ln2b_ = row(ln2_g), row(ln2_b)
  b1_, b2_, b3_ = row(b1), row(b2), row(b3)

  hist = deg_kernel(dst3d)
  hist_t = jnp.swapaxes(hist, 0, 1)

  hws0, dis = _enc_call(x_pad, W_in.astype(f32), b_in_, Wg0.astype(f32),
                        hist_t)
  acc0 = edge_kernel(hws0, src3, dst3)
  hws1 = _mid_call(acc0, hws0, dis, bg0_, ln0g_, ln0b_, Wg1.astype(f32))
  acc1 = edge_kernel(hws1, src3, dst3)
  out = _tail_call(acc1, hws1, dis, bg1_, ln1g_, ln1b_,
                   W_out.astype(f32), b_out_, ln2g_, ln2b_,
                   W1.astype(f32), b1_, W2.astype(f32).T, b2_,
                   W3.astype(f32).T, b3_)
  return out[:N, 0].astype(jnp.float64)


# speedup vs baseline: 1.1114x; 1.1114x over previous
import functools

import jax
import jax.numpy as jnp
from jax import lax
from jax.experimental import pallas as pl
from jax.experimental.pallas import tpu as pltpu
from jax.experimental.pallas import tpu_sc as plsc

N = 10000
E = 320000
D = 128
HD = D // 2
N_PAD = 10240
NC, NS = 2, 16
NW = NC * NS
B = 80
E_PAD = 320000
EPT = E_PAD // NS
NB = EPT // B
NB2 = E_PAD // NW // B
NSLOT = 4
RPT = N_PAD // NS
R = 2048
GRID = N_PAD // R

f32 = jnp.float32


@functools.lru_cache(maxsize=None)
def _sc_kernels():
  mesh = plsc.VectorSubcoreMesh(core_axis_name="c", subcore_axis_name="s")

  @functools.partial(
      pl.kernel,
      out_type=jax.ShapeDtypeStruct((NW, N_PAD), f32),
      mesh=mesh,
      scratch_types=[
          pltpu.VMEM((NB2, B), jnp.int32),
          pltpu.VMEM((N_PAD,), f32),
      ],
      compiler_params=pltpu.CompilerParams(needs_layout_passes=False),
  )
  def deg_kernel(dst_hbm, out_hbm, dbuf, hist):
    cid = lax.axis_index("c")
    sid = lax.axis_index("s")
    gid = cid * NS + sid
    pltpu.sync_copy(dst_hbm.at[gid], dbuf)
    zv = jnp.zeros((16,), f32)

    def zbody(i, c):
      hist[pl.ds(pl.multiple_of(i * 16, 16), 16)] = zv
      return c

    lax.fori_loop(jnp.int32(0), jnp.int32(N_PAD // 16), zbody, jnp.int32(0))
    ones = jnp.ones((16,), f32)

    def ebody(j, c):
      def kbody(k, c2):
        idx = dbuf[j, pl.ds(pl.multiple_of(k * 16, 16), 16)]
        plsc.addupdate_scatter(hist, [idx], ones)
        return c2

      return lax.fori_loop(jnp.int32(0), jnp.int32(B // 16), kbody, c)

    lax.fori_loop(jnp.int32(0), jnp.int32(NB2), ebody, jnp.int32(0))
    pltpu.sync_copy(hist, out_hbm.at[gid])

  @functools.partial(
      pl.kernel,
      out_type=jax.ShapeDtypeStruct((NC, N_PAD, HD), f32),
      mesh=mesh,
      scratch_types=[
          pltpu.VMEM((NB, B), jnp.int32),
          pltpu.VMEM((NB, B), jnp.int32),
          [pltpu.VMEM((B, HD), f32)] * 2,
          pltpu.VMEM_SHARED((N_PAD, HD), f32),
          [pltpu.SemaphoreType.DMA] * 2,
      ],
      compiler_params=pltpu.CompilerParams(use_tc_tiling_on_sc=False),
  )
  def edge_kernel(hws_hbm, src_hbm, dst_hbm, out,
                  sbuf, dbuf, rows, acc_sh, gsem):
    cid = lax.axis_index("c")
    sid = lax.axis_index("s")
    row0 = sid * RPT

    zv = jnp.zeros((16,), f32)

    def zbody(i, c):
      r = i // (HD // 16)
      col = (i % (HD // 16)) * 16
      rows[0][r, pl.ds(pl.multiple_of(col, 16), 16)] = zv
      return c

    lax.fori_loop(jnp.int32(0), jnp.int32(B * HD // 16), zbody, jnp.int32(0))
    for k in range(RPT // B):
      pltpu.sync_copy(rows[0], acc_sh.at[pl.ds(row0 + k * B, B)])

    pltpu.sync_copy(src_hbm.at[sid], sbuf)
    pltpu.sync_copy(dst_hbm.at[sid], dbuf)
    plsc.subcore_barrier()

    table = hws_hbm.at[cid]

    def gather(j, k):
      j = jnp.int32(j)
      return pltpu.make_async_copy(table.at[sbuf.at[j]], rows[k], gsem[k])

    def scatter(j, k):
      j = jnp.int32(j)
      pltpu.sync_copy(rows[k], acc_sh.at[dbuf.at[j]], add=True)

    gather(0, 0).start()

    def pbody(i, c):
      ja = i * 2
      jb = ja + 1
      gather(jb, 1).start()
      gather(ja, 0).wait()
      scatter(ja, 0)
      gather(ja + 2, 0).start()
      gather(jb, 1).wait()
      scatter(jb, 1)
      return c

    lax.fori_loop(jnp.int32(0), jnp.int32((NB - 2) // 2), pbody, jnp.int32(0))
    gather(NB - 1, 1).start()
    gather(NB - 2, 0).wait()
    scatter(NB - 2, 0)
    gather(NB - 1, 1).wait()
    scatter(NB - 1, 1)

    plsc.subcore_barrier()
    pltpu.sync_copy(acc_sh.at[pl.ds(row0, RPT)], out.at[cid, pl.ds(row0, RPT)])

  return deg_kernel, edge_kernel


def _ln(t, g, b):
  ones = jnp.ones((D, 1), f32)
  m = _dot(t, ones) * (1.0 / D)
  c = t - m
  v = _dot(c * c, ones) * (1.0 / D)
  return c * lax.rsqrt(v + 1e-5) * g + b


def _elu(t):
  return jnp.where(t > 0, t, jnp.exp(jnp.minimum(t, 0.0)) - 1.0)


def _dot(a, b):
  return jnp.dot(a, b, preferred_element_type=f32,
                 precision=lax.Precision.HIGHEST)


def _split_store(out_ref, t):
  out_ref[0] = t[:, :HD]
  out_ref[1] = t[:, HD:]


def _unsplit(ref):
  return jnp.concatenate([ref[0], ref[1]], axis=-1)


def _enc_body(x_ref, win_ref, bin_ref, wg0_ref, hist_ref, hws_ref, dis_ref):
  h = jnp.maximum(_dot(x_ref[...], win_ref[...]) + bin_ref[...], 0.0)
  deg = jnp.sum(hist_ref[...], axis=1, keepdims=True) + 1.0
  dis = lax.rsqrt(deg)
  dis_ref[...] = dis
  _split_store(hws_ref, _dot(h, wg0_ref[...]) * dis)


def _mid_body(acc_ref, hws_ref, dis_ref, bg_ref, lng_ref, lnb_ref,
              wg1_ref, out_ref):
  dis = dis_ref[...]
  t = (_unsplit(acc_ref) + _unsplit(hws_ref)) * dis + bg_ref[...]
  h = _elu(_ln(t, lng_ref[...], lnb_ref[...]))
  _split_store(out_ref, _dot(h, wg1_ref[...]) * dis)


def _tail_body(acc_ref, hws_ref, dis_ref, bg_ref, lng_ref, lnb_ref,
               wout_ref, bout_ref, ln2g_ref, ln2b_ref, w1_ref, b1_ref,
               w2t_ref, b2_ref, w3t_ref, b3_ref, out_ref):
  t = (_unsplit(acc_ref) + _unsplit(hws_ref)) * dis_ref[...] + bg_ref[...]
  h = _elu(_ln(t, lng_ref[...], lnb_ref[...]))
  ne = _ln(jnp.maximum(_dot(h, wout_ref[...]) + bout_ref[...], 0.0),
           ln2g_ref[...], ln2b_ref[...])
  y = jnp.maximum(_dot(ne, w1_ref[...]) + b1_ref[...], 0.0)
  w2t = w2t_ref[...]
  cols = [jnp.sum(y * w2t[k:k + 1, :], axis=1, keepdims=True)
          for k in range(w2t.shape[0])]
  y = jnp.maximum(jnp.concatenate(cols, axis=1) + b2_ref[...], 0.0)
  out_ref[...] = jnp.sum(y * w3t_ref[...], axis=1, keepdims=True) + b3_ref[...]


def _row_spec(w):
  return pl.BlockSpec((R, w), lambda i: (i, jnp.int32(0)))


def _split_spec():
  return pl.BlockSpec((2, R, HD), lambda i: (jnp.int32(0), i, jnp.int32(0)))


def _full_spec(h, w):
  return pl.BlockSpec((h, w), lambda i: (jnp.int32(0), jnp.int32(0)))


_SPLIT_SHAPE = jax.ShapeDtypeStruct((2, N_PAD, HD), f32)


def _enc_call(x_pad, w_in, b_in, wg0, hist_t):
  return pl.pallas_call(
      _enc_body,
      grid=(GRID,),
      in_specs=[
          _row_spec(D), _full_spec(D, D), _full_spec(1, D), _full_spec(D, D),
          _row_spec(NW),
      ],
      out_specs=[_split_spec(), _row_spec(1)],
      out_shape=[_SPLIT_SHAPE, jax.ShapeDtypeStruct((N_PAD, 1), f32)],
  )(x_pad, w_in, b_in, wg0, hist_t)


def _mid_call(acc, hws, dis, bg, lng, lnb, wg1):
  return pl.pallas_call(
      _mid_body,
      grid=(GRID,),
      in_specs=[
          _split_spec(), _split_spec(), _row_spec(1),
          _full_spec(1, D), _full_spec(1, D), _full_spec(1, D),
          _full_spec(D, D),
      ],
      out_specs=_split_spec(),
      out_shape=_SPLIT_SHAPE,
  )(acc, hws, dis, bg, lng, lnb, wg1)


def _tail_call(acc, hws, dis, bg, lng, lnb, wout, bout, ln2g, ln2b,
               w1, b1, w2t, b2, w3t, b3):
  h1 = w1.shape[1]
  h2 = w2t.shape[0]
  return pl.pallas_call(
      _tail_body,
      grid=(GRID,),
      in_specs=[
          _split_spec(), _split_spec(), _row_spec(1),
          _full_spec(1, D), _full_spec(1, D), _full_spec(1, D),
          _full_spec(D, D), _full_spec(1, D),
          _full_spec(1, D), _full_spec(1, D),
          _full_spec(D, h1), _full_spec(1, h1),
          _full_spec(h2, h1), _full_spec(1, h2),
          _full_spec(1, h2), _full_spec(1, 1),
      ],
      out_specs=_row_spec(1),
      out_shape=jax.ShapeDtypeStruct((N_PAD, 1), f32),
  )(acc, hws, dis, bg, lng, lnb, wout, bout, ln2g, ln2b,
    w1, b1, w2t, b2, w3t, b3)


def kernel(x, edge_index, W_in, b_in, Wg0, bg0, ln0_g, ln0_b, Wg1, bg1,
           ln1_g, ln1_b, W_out, b_out, ln2_g, ln2_b, W1, b1, W2, b2, W3, b3):
  deg_kernel, edge_kernel = _sc_kernels()

  ei = edge_index.astype(jnp.int32)
  npad = E_PAD - E
  src_p = jnp.concatenate([ei[0], jnp.zeros((npad,), jnp.int32)])
  dst_p = jnp.concatenate([ei[1], jnp.full((npad,), N_PAD - 1, jnp.int32)])
  src3 = src_p.reshape(NS, NB, B)
  dst3 = dst_p.reshape(NS, NB, B)
  dst3d = dst_p.reshape(NW, NB2, B)
  x_pad = jnp.pad(x.astype(f32), ((0, N_PAD - N), (0, 0)))

  row = lambda a: a.astype(f32).reshape(1, -1)
  b_in_, bg0_, bg1_, b_out_ = row(b_in), row(bg0), row(bg1), row(b_out)
  ln0g_, ln0b_ = row(ln0_g), row(ln0_b)
  ln1g_, ln1b_ = row(ln1_g), row(ln1_b)
  ln2g_, ln2b_ = row(ln2_g), row(ln2_b)
  b1_, b2_, b3_ = row(b1), row(b2), row(b3)

  hist = deg_kernel(dst3d)
  hist_t = jnp.swapaxes(hist, 0, 1)

  hws0, dis = _enc_call(x_pad, W_in.astype(f32), b_in_, Wg0.astype(f32),
                        hist_t)
  acc0 = edge_kernel(hws0, src3, dst3)
  hws1 = _mid_call(acc0, hws0, dis, bg0_, ln0g_, ln0b_, Wg1.astype(f32))
  acc1 = edge_kernel(hws1, src3, dst3)
  out = _tail_call(acc1, hws1, dis, bg1_, ln1g_, ln1b_,
                   W_out.astype(f32), b_out_, ln2g_, ln2b_,
                   W1.astype(f32), b1_, W2.astype(f32).T, b2_,
                   W3.astype(f32).T, b3_)
  return out[:N, 0].astype(jnp.float64)


# speedup vs baseline: 1.2033x; 1.0827x over previous
import functools

import jax
import jax.numpy as jnp
from jax import lax
from jax.experimental import pallas as pl
from jax.experimental.pallas import tpu as pltpu
from jax.experimental.pallas import tpu_sc as plsc

N = 10000
E = 320000
D = 128
HD = D // 2
N_PAD = 10240
NC, NS = 2, 16
NW = NC * NS
B = 80
E_PAD = 320000
EPT = E_PAD // NS
NB = EPT // B
NB2 = E_PAD // NW // B
NSLOT = 4
RPT = N_PAD // NS
R = 2048
GRID = N_PAD // R

f32 = jnp.float32


@functools.lru_cache(maxsize=None)
def _sc_kernels():
  mesh = plsc.VectorSubcoreMesh(core_axis_name="c", subcore_axis_name="s")

  @functools.partial(
      pl.kernel,
      out_type=jax.ShapeDtypeStruct((NW, N_PAD), f32),
      mesh=mesh,
      scratch_types=[
          pltpu.VMEM((NB2, B), jnp.int32),
          pltpu.VMEM((N_PAD,), f32),
      ],
      compiler_params=pltpu.CompilerParams(needs_layout_passes=False),
  )
  def deg_kernel(dst_hbm, out_hbm, dbuf, hist):
    cid = lax.axis_index("c")
    sid = lax.axis_index("s")
    gid = cid * NS + sid
    pltpu.sync_copy(dst_hbm.at[gid], dbuf)
    zv = jnp.zeros((16,), f32)

    def zbody(i, c):
      hist[pl.ds(pl.multiple_of(i * 16, 16), 16)] = zv
      return c

    lax.fori_loop(jnp.int32(0), jnp.int32(N_PAD // 16), zbody, jnp.int32(0))
    ones = jnp.ones((16,), f32)

    def ebody(j, c):
      def kbody(k, c2):
        idx = dbuf[j, pl.ds(pl.multiple_of(k * 16, 16), 16)]
        plsc.addupdate_scatter(hist, [idx], ones)
        return c2

      return lax.fori_loop(jnp.int32(0), jnp.int32(B // 16), kbody, c)

    lax.fori_loop(jnp.int32(0), jnp.int32(NB2), ebody, jnp.int32(0))
    pltpu.sync_copy(hist, out_hbm.at[gid])

  @functools.partial(
      pl.kernel,
      out_type=jax.ShapeDtypeStruct((NC, N_PAD, HD), f32),
      mesh=mesh,
      scratch_types=[
          pltpu.VMEM((NB, B), jnp.int32),
          pltpu.VMEM((NB, B), jnp.int32),
          [pltpu.VMEM((B, HD), f32)] * 2,
          pltpu.VMEM_SHARED((N_PAD, HD), f32),
          [pltpu.SemaphoreType.DMA] * 2,
      ],
      compiler_params=pltpu.CompilerParams(use_tc_tiling_on_sc=False),
  )
  def edge_kernel(hws_hbm, src_hbm, dst_hbm, out,
                  sbuf, dbuf, rows, acc_sh, gsem):
    cid = lax.axis_index("c")
    sid = lax.axis_index("s")
    row0 = sid * RPT

    zv = jnp.zeros((16,), f32)

    def zbody(i, c):
      r = i // (HD // 16)
      col = (i % (HD // 16)) * 16
      rows[0][r, pl.ds(pl.multiple_of(col, 16), 16)] = zv
      return c

    lax.fori_loop(jnp.int32(0), jnp.int32(B * HD // 16), zbody, jnp.int32(0))
    for k in range(RPT // B):
      pltpu.sync_copy(rows[0], acc_sh.at[pl.ds(row0 + k * B, B)])

    pltpu.sync_copy(src_hbm.at[sid], sbuf)
    pltpu.sync_copy(dst_hbm.at[sid], dbuf)
    plsc.subcore_barrier()

    table = hws_hbm.at[cid]

    def gather(j, k):
      j = jnp.int32(j)
      return pltpu.make_async_copy(table.at[sbuf.at[j]], rows[k], gsem[k])

    def scatter(j, k):
      j = jnp.int32(j)
      pltpu.sync_copy(rows[k], acc_sh.at[dbuf.at[j]], add=True)

    gather(0, 0).start()

    def pbody(i, c):
      ja = i * 2
      jb = ja + 1
      gather(jb, 1).start()
      gather(ja, 0).wait()
      scatter(ja, 0)
      gather(ja + 2, 0).start()
      gather(jb, 1).wait()
      scatter(jb, 1)
      return c

    lax.fori_loop(jnp.int32(0), jnp.int32((NB - 2) // 2), pbody, jnp.int32(0))
    gather(NB - 1, 1).start()
    gather(NB - 2, 0).wait()
    scatter(NB - 2, 0)
    gather(NB - 1, 1).wait()
    scatter(NB - 1, 1)

    plsc.subcore_barrier()
    pltpu.sync_copy(acc_sh.at[pl.ds(row0, RPT)], out.at[cid, pl.ds(row0, RPT)])

  return deg_kernel, edge_kernel


def _ln(t, g, b):
  m = jnp.mean(t, axis=-1, keepdims=True)
  c = t - m
  v = jnp.mean(c * c, axis=-1, keepdims=True)
  return c * lax.rsqrt(v + 1e-5) * g + b


def _elu(t):
  return jnp.where(t > 0, t, jnp.exp(jnp.minimum(t, 0.0)) - 1.0)


def _dot(a, b):
  return jnp.dot(a, b, preferred_element_type=f32,
                 precision=lax.Precision.HIGHEST)


def _split_store(out_ref, t):
  out_ref[0] = t[:, :HD]
  out_ref[1] = t[:, HD:]


def _unsplit(ref):
  return jnp.concatenate([ref[0], ref[1]], axis=-1)


def _enc_body(x_ref, win_ref, bin_ref, wg0_ref, hist_ref, hws_ref, dis_ref):
  h = jnp.maximum(_dot(x_ref[...], win_ref[...]) + bin_ref[...], 0.0)
  deg = jnp.sum(hist_ref[...], axis=1, keepdims=True) + 1.0
  dis = lax.rsqrt(deg)
  dis_ref[...] = dis
  _split_store(hws_ref, _dot(h, wg0_ref[...]) * dis)


def _mid_body(acc_ref, hws_ref, dis_ref, bg_ref, lng_ref, lnb_ref,
              wg1_ref, out_ref):
  dis = dis_ref[...]
  t = (_unsplit(acc_ref) + _unsplit(hws_ref)) * dis + bg_ref[...]
  h = _elu(_ln(t, lng_ref[...], lnb_ref[...]))
  _split_store(out_ref, _dot(h, wg1_ref[...]) * dis)


def _tail_body(acc_ref, hws_ref, dis_ref, bg_ref, lng_ref, lnb_ref,
               wout_ref, bout_ref, ln2g_ref, ln2b_ref, w1_ref, b1_ref,
               w2t_ref, b2_ref, w3t_ref, b3_ref, out_ref):
  t = (_unsplit(acc_ref) + _unsplit(hws_ref)) * dis_ref[...] + bg_ref[...]
  h = _elu(_ln(t, lng_ref[...], lnb_ref[...]))
  ne = _ln(jnp.maximum(_dot(h, wout_ref[...]) + bout_ref[...], 0.0),
           ln2g_ref[...], ln2b_ref[...])
  y = jnp.maximum(_dot(ne, w1_ref[...]) + b1_ref[...], 0.0)
  w2t = w2t_ref[...]
  cols = [jnp.sum(y * w2t[k:k + 1, :], axis=1, keepdims=True)
          for k in range(w2t.shape[0])]
  y = jnp.maximum(jnp.concatenate(cols, axis=1) + b2_ref[...], 0.0)
  out_ref[...] = jnp.sum(y * w3t_ref[...], axis=1, keepdims=True) + b3_ref[...]


def _row_spec(w):
  return pl.BlockSpec((R, w), lambda i: (i, jnp.int32(0)))


def _split_spec():
  return pl.BlockSpec((2, R, HD), lambda i: (jnp.int32(0), i, jnp.int32(0)))


def _full_spec(h, w):
  return pl.BlockSpec((h, w), lambda i: (jnp.int32(0), jnp.int32(0)))


_SPLIT_SHAPE = jax.ShapeDtypeStruct((2, N_PAD, HD), f32)


def _enc_call(x_pad, w_in, b_in, wg0, hist_t):
  return pl.pallas_call(
      _enc_body,
      grid=(GRID,),
      in_specs=[
          _row_spec(D), _full_spec(D, D), _full_spec(1, D), _full_spec(D, D),
          _row_spec(NW),
      ],
      out_specs=[_split_spec(), _row_spec(1)],
      out_shape=[_SPLIT_SHAPE, jax.ShapeDtypeStruct((N_PAD, 1), f32)],
  )(x_pad, w_in, b_in, wg0, hist_t)


def _mid_call(acc, hws, dis, bg, lng, lnb, wg1):
  return pl.pallas_call(
      _mid_body,
      grid=(GRID,),
      in_specs=[
          _split_spec(), _split_spec(), _row_spec(1),
          _full_spec(1, D), _full_spec(1, D), _full_spec(1, D),
          _full_spec(D, D),
      ],
      out_specs=_split_spec(),
      out_shape=_SPLIT_SHAPE,
  )(acc, hws, dis, bg, lng, lnb, wg1)


def _tail_call(acc, hws, dis, bg, lng, lnb, wout, bout, ln2g, ln2b,
               w1, b1, w2t, b2, w3t, b3):
  h1 = w1.shape[1]
  h2 = w2t.shape[0]
  return pl.pallas_call(
      _tail_body,
      grid=(GRID,),
      in_specs=[
          _split_spec(), _split_spec(), _row_spec(1),
          _full_spec(1, D), _full_spec(1, D), _full_spec(1, D),
          _full_spec(D, D), _full_spec(1, D),
          _full_spec(1, D), _full_spec(1, D),
          _full_spec(D, h1), _full_spec(1, h1),
          _full_spec(h2, h1), _full_spec(1, h2),
          _full_spec(1, h2), _full_spec(1, 1),
      ],
      out_specs=_row_spec(1),
      out_shape=jax.ShapeDtypeStruct((N_PAD, 1), f32),
  )(acc, hws, dis, bg, lng, lnb, wout, bout, ln2g, ln2b,
    w1, b1, w2t, b2, w3t, b3)


def kernel(x, edge_index, W_in, b_in, Wg0, bg0, ln0_g, ln0_b, Wg1, bg1,
           ln1_g, ln1_b, W_out, b_out, ln2_g, ln2_b, W1, b1, W2, b2, W3, b3):
  deg_kernel, edge_kernel = _sc_kernels()

  ei = edge_index.astype(jnp.int32)
  npad = E_PAD - E
  src_p = jnp.concatenate([ei[0], jnp.zeros((npad,), jnp.int32)])
  dst_p = jnp.concatenate([ei[1], jnp.full((npad,), N_PAD - 1, jnp.int32)])
  src3 = src_p.reshape(NS, NB, B)
  dst3 = dst_p.reshape(NS, NB, B)
  dst3d = dst_p.reshape(NW, NB2, B)
  x_pad = jnp.pad(x.astype(f32), ((0, N_PAD - N), (0, 0)))

  row = lambda a: a.astype(f32).reshape(1, -1)
  b_in_, bg0_, bg1_, b_out_ = row(b_in), row(bg0), row(bg1), row(b_out)
  ln0g_, ln0b_ = row(ln0_g), row(ln0_b)
  ln1g_, ln1b_ = row(ln1_g), row(ln1_b)
  ln2g_, ln2b_ = row(ln2_g), row(ln2_b)
  b1_, b2_, b3_ = row(b1), row(b2), row(b3)

  hist = deg_kernel(dst3d)
  hist_t = jnp.swapaxes(hist, 0, 1)

  hws0, dis = _enc_call(x_pad, W_in.astype(f32), b_in_, Wg0.astype(f32),
                        hist_t)
  acc0 = edge_kernel(hws0, src3, dst3)
  hws1 = _mid_call(acc0, hws0, dis, bg0_, ln0g_, ln0b_, Wg1.astype(f32))
  acc1 = edge_kernel(hws1, src3, dst3)
  out = _tail_call(acc1, hws1, dis, bg1_, ln1g_, ln1b_,
                   W_out.astype(f32), b_out_, ln2g_, ln2b_,
                   W1.astype(f32), b1_, W2.astype(f32).T, b2_,
                   W3.astype(f32).T, b3_)
  return out[:N, 0].astype(jnp.float64)
